# Initial kernel scaffold; baseline (speedup 1.0000x reference)
import jax
import jax.numpy as jnp
from jax.experimental import pallas as pl


def kernel(input, edge_index, edge_weight, W):
    raise NotImplementedError("write your pallas kernel here")


# speedup vs baseline: 4.4251x; 4.4251x over previous
import functools

import jax
import jax.numpy as jnp
from jax import lax
from jax.experimental import pallas as pl
from jax.experimental.pallas import tpu as pltpu
from jax.experimental.pallas import tpu_sc as plsc

N = 10000
E = 320000
D = 128

NC = 2
NS = 16
NW = NC * NS
EP = E // NW
C = 80
NCHUNK = EP // C
ROWS_PER_TILE = 624
TAIL0 = NS * ROWS_PER_TILE
TAIL = N - TAIL0


def _mm_block(x_ref, w_ref, o_ref):
    o_ref[...] = jnp.dot(x_ref[...], w_ref[...],
                         preferred_element_type=jnp.float32)


def _matmul(x, w):
    grid = 10
    bn = N // grid
    return pl.pallas_call(
        _mm_block,
        grid=(grid,),
        in_specs=[
            pl.BlockSpec((bn, D), lambda i: (i, 0)),
            pl.BlockSpec((D, D), lambda i: (0, 0)),
        ],
        out_specs=pl.BlockSpec((bn, D), lambda i: (i, 0)),
        out_shape=jax.ShapeDtypeStruct((N, D), jnp.float32),
    )(x, w)


def _combine_block(p_ref, o_ref):
    o_ref[...] = jnp.maximum(p_ref[0] + p_ref[1], 0.0)


def _combine(partials):
    grid = 10
    bn = N // grid
    return pl.pallas_call(
        _combine_block,
        grid=(grid,),
        in_specs=[pl.BlockSpec((NC, bn, D), lambda i: (0, i, 0))],
        out_specs=pl.BlockSpec((bn, D), lambda i: (i, 0)),
        out_shape=jax.ShapeDtypeStruct((N, D), jnp.float32),
    )(partials)


def _sc_body(support_hbm, src_hbm, dst_hbm, w_hbm, zeros_hbm, out_hbm,
             src_v, dst_v, w_v, rows_v, acc, gsem):
    c = lax.axis_index("c")
    s = lax.axis_index("s")
    wid = s * NC + c
    base = wid * EP

    row0 = s * ROWS_PER_TILE
    pltpu.sync_copy(zeros_hbm.at[pl.ds(row0, ROWS_PER_TILE)],
                    acc.at[pl.ds(row0, ROWS_PER_TILE)])

    @pl.when(s == NS - 1)
    def _zero_tail():
        pltpu.sync_copy(zeros_hbm.at[pl.ds(TAIL0, TAIL)],
                        acc.at[pl.ds(TAIL0, TAIL)])

    plsc.subcore_barrier()

    def chunk_body(k, carry):
        off = base + k * C
        pltpu.sync_copy(src_hbm.at[pl.ds(off, C)], src_v)
        pltpu.sync_copy(dst_hbm.at[pl.ds(off, C)], dst_v)
        pltpu.sync_copy(w_hbm.at[pl.ds(off, C)], w_v)
        pltpu.async_copy(support_hbm.at[src_v], rows_v, gsem).wait()

        dnums = lax.GatherDimensionNumbers(
            offset_dims=(), collapsed_slice_dims=(0,), start_index_map=(0,))

        def scale_group(g, carry2):
            w16 = w_v[pl.ds(g * 16, 16)]
            for i in range(16):
                wb = lax.gather(
                    w16, jnp.full((16, 1), i, jnp.int32), dnums,
                    slice_sizes=(1,),
                    mode=lax.GatherScatterMode.PROMISE_IN_BOUNDS)
                r = g * 16 + i
                for d in range(D // 16):
                    sl = pl.ds(d * 16, 16)
                    rows_v[r, sl] = rows_v[r, sl] * wb
            return carry2

        lax.fori_loop(0, C // 16, scale_group, 0, unroll=False)

        pltpu.sync_copy(rows_v, acc.at[dst_v], add=True)
        return carry

    lax.fori_loop(0, NCHUNK, chunk_body, 0, unroll=False)

    plsc.subcore_barrier()
    pltpu.sync_copy(acc.at[pl.ds(row0, ROWS_PER_TILE)],
                    out_hbm.at[c, pl.ds(row0, ROWS_PER_TILE)])

    @pl.when(s == NS - 1)
    def _read_tail():
        pltpu.sync_copy(acc.at[pl.ds(TAIL0, TAIL)],
                        out_hbm.at[c, pl.ds(TAIL0, TAIL)])


def _sc_spmm(support, src, dst, w, zeros):
    mesh = plsc.VectorSubcoreMesh(core_axis_name="c", subcore_axis_name="s")
    f = pl.kernel(
        _sc_body,
        out_type=jax.ShapeDtypeStruct((NC, N, D), jnp.float32),
        mesh=mesh,
        scratch_types=[
            pltpu.VMEM((C,), jnp.int32),
            pltpu.VMEM((C,), jnp.int32),
            pltpu.VMEM((C,), jnp.float32),
            pltpu.VMEM((C, D), jnp.float32),
            pltpu.VMEM_SHARED((N, D), jnp.float32),
            pltpu.SemaphoreType.DMA,
        ],
    )
    return f(support, src, dst, w, zeros)


def kernel(input, edge_index, edge_weight, W):
    src = edge_index[0].astype(jnp.int32)
    dst = edge_index[1].astype(jnp.int32)
    support = _matmul(input, W)
    zeros = jnp.zeros((N, D), jnp.float32)
    partials = _sc_spmm(support, src, dst, edge_weight, zeros)
    return _combine(partials)
